# baseline (speedup 1.0000x reference)
import functools

import jax
import jax.numpy as jnp
from jax import lax
from jax.experimental import pallas as pl
from jax.experimental.pallas import tpu as pltpu
from jax.experimental.pallas import tpu_sc as plsc

N = 10000
E = 320000
M = 128
HID = 16
HEADS = 8
MY = 64

ROW_BLOCK = 1000

NPAD = 10112
EBLK = 128
NWORK = 32
NBLK = -(-E // (NWORK * EBLK))
EPAD = NWORK * EBLK * NBLK
NROW = NPAD // 16


def _pre_body(x_ref, w_ref, asdw_ref, hx_ref):
    h = x_ref[...] @ w_ref[...]
    hx_ref[...] = jnp.concatenate([h, h @ asdw_ref[...]], axis=1)


def _pre_call(x, W, ASDW):
    n, m = x.shape
    k = W.shape[1]
    cp = k + ASDW.shape[1]
    grid = n // ROW_BLOCK
    return pl.pallas_call(
        _pre_body,
        grid=(grid,),
        in_specs=[
            pl.BlockSpec((ROW_BLOCK, m), lambda i: (i, 0)),
            pl.BlockSpec((m, k), lambda i: (0, 0)),
            pl.BlockSpec((k, ASDW.shape[1]), lambda i: (0, 0)),
        ],
        out_specs=pl.BlockSpec((ROW_BLOCK, cp), lambda i: (i, 0)),
        out_shape=jax.ShapeDtypeStruct((n, cp), jnp.float32),
    )(x, W, ASDW)


def _mid_body(acca_ref, accb_ref, rda_ref, rdb_ref, b_ref, w_ref, asdw_ref,
              hx_ref):
    blka = acca_ref[0] + acca_ref[1]
    blkb = accb_ref[0] + accb_ref[1]
    c = rda_ref.shape[1]
    den = blka @ rda_ref[...] + blkb @ rdb_ref[...]
    num = jnp.concatenate([blka[:, : c // 2], blkb[:, : c // 2]], axis=1)
    act = num / (den + 1e-16) + b_ref[...]
    act = jnp.where(act > 0, act, jnp.exp(jnp.minimum(act, 0.0)) - 1.0)
    h = act @ w_ref[...]
    hx_ref[...] = jnp.concatenate([h, h @ asdw_ref[...]], axis=1)


def _mid_call(acca, accb, RDA, RDB, b, W, ASDW):
    _, n, cp = acca.shape
    k = W.shape[1]
    kp = k + ASDW.shape[1]
    grid = n // ROW_BLOCK
    return pl.pallas_call(
        _mid_body,
        grid=(grid,),
        in_specs=[
            pl.BlockSpec((2, ROW_BLOCK, cp), lambda i: (0, i, 0)),
            pl.BlockSpec((2, ROW_BLOCK, cp), lambda i: (0, i, 0)),
            pl.BlockSpec((cp, RDA.shape[1]), lambda i: (0, 0)),
            pl.BlockSpec((cp, RDA.shape[1]), lambda i: (0, 0)),
            pl.BlockSpec((1, RDA.shape[1]), lambda i: (0, 0)),
            pl.BlockSpec((RDA.shape[1], k), lambda i: (0, 0)),
            pl.BlockSpec((k, ASDW.shape[1]), lambda i: (0, 0)),
        ],
        out_specs=pl.BlockSpec((ROW_BLOCK, kp), lambda i: (i, 0)),
        out_shape=jax.ShapeDtypeStruct((n, kp), jnp.float32),
    )(acca, accb, RDA, RDB, b, W, ASDW)


def _post_body(acc_ref, repd_ref, b_ref, out_ref):
    blk = acc_ref[0] + acc_ref[1]
    c = repd_ref.shape[1]
    den = blk @ repd_ref[...]
    out_ref[...] = blk[:, :c] / (den + 1e-16) + b_ref[...]


def _post_call(acc, REPD, b):
    _, n, cp = acc.shape
    c = REPD.shape[1]
    grid = n // ROW_BLOCK
    return pl.pallas_call(
        _post_body,
        grid=(grid,),
        in_specs=[
            pl.BlockSpec((2, ROW_BLOCK, cp), lambda i: (0, i, 0)),
            pl.BlockSpec((cp, c), lambda i: (0, 0)),
            pl.BlockSpec((1, c), lambda i: (0, 0)),
        ],
        out_specs=pl.BlockSpec((ROW_BLOCK, c), lambda i: (i, 0)),
        out_shape=jax.ShapeDtypeStruct((n, c), jnp.float32),
    )(acc, REPD, b)


def _make_edge_kernel(H, C):
    CH = C // H
    CP = C + 16
    mesh = plsc.VectorSubcoreMesh(core_axis_name="c", subcore_axis_name="s")

    @functools.partial(
        pl.kernel,
        out_type=jax.ShapeDtypeStruct((2, NPAD, CP), jnp.float32),
        mesh=mesh,
        scratch_types=[
            pltpu.VMEM((NBLK, EBLK), jnp.int32),
            pltpu.VMEM((NBLK, EBLK), jnp.int32),
            pltpu.VMEM((2, EBLK, CP), jnp.float32),
            pltpu.VMEM((2, EBLK, 16), jnp.float32),
            pltpu.VMEM((2, EBLK, CP), jnp.float32),
            pltpu.VMEM_SHARED((NPAD, CP), jnp.float32),
            pltpu.SemaphoreType.DMA,
            pltpu.SemaphoreType.DMA,
            pltpu.SemaphoreType.DMA,
            pltpu.SemaphoreType.DMA,
            pltpu.SemaphoreType.DMA,
            pltpu.SemaphoreType.DMA,
        ],
        compiler_params=pltpu.CompilerParams(
            needs_layout_passes=False, use_tc_tiling_on_sc=False),
    )
    def ek(src_hbm, dst_hbm, hx_hbm, att_hbm, zacc_hbm, acc_out,
           idx_s, idx_d, hxb, attb, msg, acc,
           sem_h0, sem_h1, sem_a0, sem_a1, sem_w0, sem_w1):
        sem_h = (sem_h0, sem_h1)
        sem_a = (sem_a0, sem_a1)
        sem_w = (sem_w0, sem_w1)
        cid = lax.axis_index("c")
        sid = lax.axis_index("s")
        wid = cid * 16 + sid

        pltpu.sync_copy(zacc_hbm.at[pl.ds(sid * NROW, NROW)],
                        acc.at[pl.ds(sid * NROW, NROW)])
        plsc.subcore_barrier()

        pltpu.sync_copy(src_hbm.at[wid], idx_s)
        pltpu.sync_copy(dst_hbm.at[wid], idx_d)

        lanes = lax.iota(jnp.int32, 16)
        zero16 = jnp.zeros((16,), jnp.float32)

        @plsc.parallel_loop(0, EBLK, unroll=4)
        def _(e):
            msg[0, e, pl.ds(C, 16)] = zero16
            msg[1, e, pl.ds(C, 16)] = zero16

        def gather(slot, b):
            pltpu.async_copy(hx_hbm.at[idx_s.at[b]], hxb.at[slot],
                             sem_h[slot])
            pltpu.async_copy(att_hbm.at[idx_d.at[b]], attb.at[slot],
                             sem_a[slot])

        def gather_wait(slot, b):
            pltpu.make_async_copy(hx_hbm.at[idx_s.at[b]], hxb.at[slot],
                                  sem_h[slot]).wait()
            pltpu.make_async_copy(att_hbm.at[idx_d.at[b]], attb.at[slot],
                                  sem_a[slot]).wait()

        def scatter(slot, b):
            pltpu.async_copy(msg.at[slot], acc.at[idx_d.at[b]],
                             sem_w[slot], add=True)

        def scatter_wait(slot, b):
            pltpu.make_async_copy(msg.at[slot], acc.at[idx_d.at[b]],
                                  sem_w[slot]).wait()

        def compute(slot):
            @plsc.parallel_loop(0, EBLK // 16, unroll=4)
            def _(g):
                row = lanes + g * 16
                for h in range(H):
                    sv = plsc.load_gather(
                        hxb.at[slot], [row, jnp.full((16,), C + h, jnp.int32)])
                    dv = plsc.load_gather(
                        attb.at[slot], [row, jnp.full((16,), 15 - h, jnp.int32)])
                    ev = sv + dv
                    w = jnp.exp(jnp.maximum(ev, 0.2 * ev))
                    plsc.store_scatter(
                        msg.at[slot], [row, jnp.full((16,), C + h, jnp.int32)], w)
                    for c in range(CH):
                        col = jnp.full((16,), h * CH + c, jnp.int32)
                        hv = plsc.load_gather(hxb.at[slot], [row, col])
                        plsc.store_scatter(msg.at[slot], [row, col], hv * w)

        gather(0, 0)

        def pair(i, _):
            b0 = 2 * i
            gather(1, b0 + 1)
            gather_wait(0, b0)

            @pl.when(i > 0)
            def _():
                scatter_wait(0, b0 - 2)

            compute(0)
            scatter(0, b0)
            gather(0, b0 + 2)
            gather_wait(1, b0 + 1)

            @pl.when(i > 0)
            def _():
                scatter_wait(1, b0 - 1)

            compute(1)
            scatter(1, b0 + 1)
            return 0

        lax.fori_loop(0, (NBLK - 1) // 2, pair, 0)
        gather_wait(0, NBLK - 1)
        scatter_wait(0, NBLK - 3)
        compute(0)
        scatter(0, NBLK - 1)
        scatter_wait(0, NBLK - 1)
        scatter_wait(1, NBLK - 2)

        plsc.subcore_barrier()
        pltpu.sync_copy(acc.at[pl.ds(sid * NROW, NROW)],
                        acc_out.at[cid, pl.ds(sid * NROW, NROW)])

    return ek


_edge_l1h = _make_edge_kernel(4, 64)
_edge_l2 = _make_edge_kernel(1, MY)


def _pad_rows(a):
    return jnp.pad(a, ((0, NPAD - N), (0, 0)))


def kernel(x, edge_index, W1, att_src1, att_dst1, b1, W2, att_src2, att_dst2, b2):
    src = edge_index[0].astype(jnp.int32)
    dst = edge_index[1].astype(jnp.int32)
    pad = jnp.full((EPAD - E,), N, jnp.int32)
    srcp = jnp.concatenate([src, pad]).reshape(NWORK, NBLK, EBLK)
    dstp = jnp.concatenate([dst, pad]).reshape(NWORK, NBLK, EBLK)

    eye8 = jnp.eye(HEADS, dtype=jnp.float32)
    AS1 = (att_src1[:, :, None] * eye8[:, None, :]).reshape(HEADS * HID, HEADS)
    AD1 = (att_dst1[:, :, None] * eye8[:, None, :]).reshape(HEADS * HID, HEADS)
    z8 = jnp.zeros((HEADS * HID, 8), jnp.float32)
    ATTA = jnp.concatenate([AS1[:, 0:4], z8, AD1[:, 3::-1]], axis=1)
    ATTB = jnp.concatenate([AS1[:, 4:8], z8, AD1[:, 7:3:-1]], axis=1)
    ASDW1 = jnp.concatenate([ATTA, ATTB], axis=1)
    ASDW2 = jnp.concatenate(
        [att_src2.T, jnp.zeros((MY, 14), jnp.float32), att_dst2.T], axis=1
    )
    REP1 = (eye8[:, :, None] * jnp.ones((1, 1, HID))).reshape(HEADS, HEADS * HID)
    z64 = jnp.zeros((MY, HEADS * HID), jnp.float32)
    z12 = jnp.zeros((12, HEADS * HID), jnp.float32)
    RDA = jnp.concatenate([z64, REP1[0:4], z12], axis=0)
    RDB = jnp.concatenate([z64, REP1[4:8], z12], axis=0)
    REPD2 = jnp.concatenate(
        [jnp.zeros((MY, MY), jnp.float32),
         jnp.zeros((16, MY), jnp.float32).at[0, :].set(1.0)], axis=0)

    zacc = jnp.zeros((NPAD, MY + 16), jnp.float32)

    hx1 = _pre_call(x, W1, ASDW1)
    hxa = _pad_rows(jnp.concatenate([hx1[:, 0:64], hx1[:, 128:144]], axis=1))
    hxb = _pad_rows(jnp.concatenate([hx1[:, 64:128], hx1[:, 144:160]], axis=1))
    acca = _edge_l1h(srcp, dstp, hxa, hxa[:, 64:], zacc)
    accb = _edge_l1h(srcp, dstp, hxb, hxb[:, 64:], zacc)
    hx2 = _mid_call(acca[:, :N], accb[:, :N], RDA, RDB, b1.reshape(1, -1),
                    W2, ASDW2)
    acc2 = _edge_l2(srcp, dstp, _pad_rows(hx2), _pad_rows(hx2[:, MY:]), zacc)
    out = _post_call(acc2[:, :N], REPD2, b2.reshape(1, -1))
    return out


# speedup vs baseline: 1.3623x; 1.3623x over previous
import functools

import jax
import jax.numpy as jnp
from jax import lax
from jax.experimental import pallas as pl
from jax.experimental.pallas import tpu as pltpu
from jax.experimental.pallas import tpu_sc as plsc

N = 10000
E = 320000
M = 128
HID = 16
HEADS = 8
MY = 64

ROW_BLOCK = 1000

NPAD = 10112
EBLK = 128
NWORK = 32
_NBLK0 = -(-E // (NWORK * EBLK))
NBLK = _NBLK0 + 1 - _NBLK0 % 2
EPAD = NWORK * EBLK * NBLK
NROW = NPAD // 16


def _pre_body(x_ref, w_ref, asdw_ref, hx_ref):
    h = x_ref[...] @ w_ref[...]
    hx_ref[...] = jnp.concatenate([h, h @ asdw_ref[...]], axis=1)


def _pre_call(x, W, ASDW):
    n, m = x.shape
    k = W.shape[1]
    cp = k + ASDW.shape[1]
    grid = n // ROW_BLOCK
    return pl.pallas_call(
        _pre_body,
        grid=(grid,),
        in_specs=[
            pl.BlockSpec((ROW_BLOCK, m), lambda i: (i, 0)),
            pl.BlockSpec((m, k), lambda i: (0, 0)),
            pl.BlockSpec((k, ASDW.shape[1]), lambda i: (0, 0)),
        ],
        out_specs=pl.BlockSpec((ROW_BLOCK, cp), lambda i: (i, 0)),
        out_shape=jax.ShapeDtypeStruct((n, cp), jnp.float32),
    )(x, W, ASDW)


def _mid_body(acca_ref, accb_ref, rda_ref, rdb_ref, b_ref, w_ref, asdw_ref,
              hx_ref):
    blka = acca_ref[0] + acca_ref[1]
    blkb = accb_ref[0] + accb_ref[1]
    c = rda_ref.shape[1]
    den = blka @ rda_ref[...] + blkb @ rdb_ref[...]
    num = jnp.concatenate([blka[:, : c // 2], blkb[:, : c // 2]], axis=1)
    act = num / (den + 1e-16) + b_ref[...]
    act = jnp.where(act > 0, act, jnp.exp(jnp.minimum(act, 0.0)) - 1.0)
    h = act @ w_ref[...]
    hx_ref[...] = jnp.concatenate([h, h @ asdw_ref[...]], axis=1)


def _mid_call(acca, accb, RDA, RDB, b, W, ASDW):
    _, n, cp = acca.shape
    k = W.shape[1]
    kp = k + ASDW.shape[1]
    grid = n // ROW_BLOCK
    return pl.pallas_call(
        _mid_body,
        grid=(grid,),
        in_specs=[
            pl.BlockSpec((2, ROW_BLOCK, cp), lambda i: (0, i, 0)),
            pl.BlockSpec((2, ROW_BLOCK, cp), lambda i: (0, i, 0)),
            pl.BlockSpec((cp, RDA.shape[1]), lambda i: (0, 0)),
            pl.BlockSpec((cp, RDA.shape[1]), lambda i: (0, 0)),
            pl.BlockSpec((1, RDA.shape[1]), lambda i: (0, 0)),
            pl.BlockSpec((RDA.shape[1], k), lambda i: (0, 0)),
            pl.BlockSpec((k, ASDW.shape[1]), lambda i: (0, 0)),
        ],
        out_specs=pl.BlockSpec((ROW_BLOCK, kp), lambda i: (i, 0)),
        out_shape=jax.ShapeDtypeStruct((n, kp), jnp.float32),
    )(acca, accb, RDA, RDB, b, W, ASDW)


def _post_body(acc_ref, repd_ref, b_ref, out_ref):
    blk = acc_ref[0] + acc_ref[1]
    c = repd_ref.shape[1]
    den = blk @ repd_ref[...]
    out_ref[...] = blk[:, :c] / (den + 1e-16) + b_ref[...]


def _post_call(acc, REPD, b):
    _, n, cp = acc.shape
    c = REPD.shape[1]
    grid = n // ROW_BLOCK
    return pl.pallas_call(
        _post_body,
        grid=(grid,),
        in_specs=[
            pl.BlockSpec((2, ROW_BLOCK, cp), lambda i: (0, i, 0)),
            pl.BlockSpec((cp, c), lambda i: (0, 0)),
            pl.BlockSpec((1, c), lambda i: (0, 0)),
        ],
        out_specs=pl.BlockSpec((ROW_BLOCK, c), lambda i: (i, 0)),
        out_shape=jax.ShapeDtypeStruct((n, c), jnp.float32),
    )(acc, REPD, b)


def _make_edge_kernel(H, C):
    CH = C // H
    CP = C + 8
    mesh = plsc.VectorSubcoreMesh(core_axis_name="c", subcore_axis_name="s")

    @functools.partial(
        pl.kernel,
        out_type=jax.ShapeDtypeStruct((2, NPAD, CP), jnp.float32),
        mesh=mesh,
        scratch_types=[
            pltpu.VMEM((NBLK, EBLK), jnp.int32),
            pltpu.VMEM((NBLK, EBLK), jnp.int32),
            pltpu.VMEM((2, EBLK, CP), jnp.float32),
            pltpu.VMEM((2, EBLK, 8), jnp.float32),
            pltpu.VMEM((2, EBLK, CP), jnp.float32),
            pltpu.VMEM_SHARED((NPAD, CP), jnp.float32),
            pltpu.SemaphoreType.DMA,
            pltpu.SemaphoreType.DMA,
            pltpu.SemaphoreType.DMA,
            pltpu.SemaphoreType.DMA,
            pltpu.SemaphoreType.DMA,
            pltpu.SemaphoreType.DMA,
        ],
        compiler_params=pltpu.CompilerParams(
            needs_layout_passes=False, use_tc_tiling_on_sc=False),
    )
    def ek(src_hbm, dst_hbm, hx_hbm, att_hbm, zacc_hbm, acc_out,
           idx_s, idx_d, hxb, attb, msg, acc,
           sem_h0, sem_h1, sem_a0, sem_a1, sem_w0, sem_w1):
        sem_h = (sem_h0, sem_h1)
        sem_a = (sem_a0, sem_a1)
        sem_w = (sem_w0, sem_w1)
        cid = lax.axis_index("c")
        sid = lax.axis_index("s")
        wid = cid * 16 + sid

        pltpu.sync_copy(zacc_hbm.at[pl.ds(sid * NROW, NROW)],
                        acc.at[pl.ds(sid * NROW, NROW)])
        plsc.subcore_barrier()

        pltpu.sync_copy(src_hbm.at[wid], idx_s)
        pltpu.sync_copy(dst_hbm.at[wid], idx_d)

        lanes = lax.iota(jnp.int32, 16)

        pltpu.sync_copy(zacc_hbm.at[pl.ds(0, EBLK)], msg.at[0])
        pltpu.sync_copy(zacc_hbm.at[pl.ds(0, EBLK)], msg.at[1])

        def gather(slot, b):
            pltpu.async_copy(hx_hbm.at[idx_s.at[b]], hxb.at[slot],
                             sem_h[slot])
            pltpu.async_copy(att_hbm.at[idx_d.at[b]], attb.at[slot],
                             sem_a[slot])

        def gather_wait(slot, b):
            pltpu.make_async_copy(hx_hbm.at[idx_s.at[b]], hxb.at[slot],
                                  sem_h[slot]).wait()
            pltpu.make_async_copy(att_hbm.at[idx_d.at[b]], attb.at[slot],
                                  sem_a[slot]).wait()

        def scatter(slot, b):
            pltpu.async_copy(msg.at[slot], acc.at[idx_d.at[b]],
                             sem_w[slot], add=True)

        def scatter_wait(slot, b):
            pltpu.make_async_copy(msg.at[slot], acc.at[idx_d.at[b]],
                                  sem_w[slot]).wait()

        def compute(slot):
            @plsc.parallel_loop(0, EBLK // 16, unroll=2)
            def _(g):
                row = lanes + g * 16
                for h in range(H):
                    sv = plsc.load_gather(
                        hxb.at[slot], [row, jnp.full((16,), C + h, jnp.int32)])
                    dv = plsc.load_gather(
                        attb.at[slot], [row, jnp.full((16,), 7 - h, jnp.int32)])
                    ev = sv + dv
                    w = jnp.exp(jnp.maximum(ev, 0.2 * ev))
                    plsc.store_scatter(
                        msg.at[slot], [row, jnp.full((16,), C + h, jnp.int32)], w)
                    for c in range(CH):
                        col = jnp.full((16,), h * CH + c, jnp.int32)
                        hv = plsc.load_gather(hxb.at[slot], [row, col])
                        plsc.store_scatter(msg.at[slot], [row, col], hv * w)

        gather(0, 0)

        def pair(i, _):
            b0 = 2 * i
            gather(1, b0 + 1)
            gather_wait(0, b0)

            @pl.when(i > 0)
            def _():
                scatter_wait(0, b0 - 2)

            compute(0)
            scatter(0, b0)
            gather(0, b0 + 2)
            gather_wait(1, b0 + 1)

            @pl.when(i > 0)
            def _():
                scatter_wait(1, b0 - 1)

            compute(1)
            scatter(1, b0 + 1)
            return 0

        lax.fori_loop(0, (NBLK - 1) // 2, pair, 0)
        gather_wait(0, NBLK - 1)
        scatter_wait(0, NBLK - 3)
        compute(0)
        scatter(0, NBLK - 1)
        scatter_wait(0, NBLK - 1)
        scatter_wait(1, NBLK - 2)

        plsc.subcore_barrier()
        pltpu.sync_copy(acc.at[pl.ds(sid * NROW, NROW)],
                        acc_out.at[cid, pl.ds(sid * NROW, NROW)])

    return ek


_edge_l1h = _make_edge_kernel(4, 64)
_edge_l2 = _make_edge_kernel(1, MY)


def _pad_rows(a):
    return jnp.pad(a, ((0, NPAD - N), (0, 0)))


def kernel(x, edge_index, W1, att_src1, att_dst1, b1, W2, att_src2, att_dst2, b2):
    src = edge_index[0].astype(jnp.int32)
    dst = edge_index[1].astype(jnp.int32)
    pad = jnp.full((EPAD - E,), N, jnp.int32)
    srcp = jnp.concatenate([src, pad]).reshape(NWORK, NBLK, EBLK)
    dstp = jnp.concatenate([dst, pad]).reshape(NWORK, NBLK, EBLK)

    eye8 = jnp.eye(HEADS, dtype=jnp.float32)
    AS1 = (att_src1[:, :, None] * eye8[:, None, :]).reshape(HEADS * HID, HEADS)
    AD1 = (att_dst1[:, :, None] * eye8[:, None, :]).reshape(HEADS * HID, HEADS)
    ATTA = jnp.concatenate([AS1[:, 0:4], AD1[:, 3::-1]], axis=1)
    ATTB = jnp.concatenate([AS1[:, 4:8], AD1[:, 7:3:-1]], axis=1)
    ASDW1 = jnp.concatenate([ATTA, ATTB], axis=1)
    ASDW2 = jnp.concatenate(
        [att_src2.T, jnp.zeros((MY, 6), jnp.float32), att_dst2.T], axis=1
    )
    REP1 = (eye8[:, :, None] * jnp.ones((1, 1, HID))).reshape(HEADS, HEADS * HID)
    z64 = jnp.zeros((MY, HEADS * HID), jnp.float32)
    z4 = jnp.zeros((4, HEADS * HID), jnp.float32)
    RDA = jnp.concatenate([z64, REP1[0:4], z4], axis=0)
    RDB = jnp.concatenate([z64, REP1[4:8], z4], axis=0)
    REPD2 = jnp.concatenate(
        [jnp.zeros((MY, MY), jnp.float32),
         jnp.zeros((8, MY), jnp.float32).at[0, :].set(1.0)], axis=0)

    zacc = jnp.zeros((NPAD, MY + 8), jnp.float32)

    hx1 = _pre_call(x, W1, ASDW1)
    hxa = _pad_rows(jnp.concatenate([hx1[:, 0:64], hx1[:, 128:136]], axis=1))
    hxb = _pad_rows(jnp.concatenate([hx1[:, 64:128], hx1[:, 136:144]], axis=1))
    acca = _edge_l1h(srcp, dstp, hxa, hxa[:, 64:], zacc)
    accb = _edge_l1h(srcp, dstp, hxb, hxb[:, 64:], zacc)
    hx2 = _mid_call(acca[:, :N], accb[:, :N], RDA, RDB, b1.reshape(1, -1),
                    W2, ASDW2)
    acc2 = _edge_l2(srcp, dstp, _pad_rows(hx2), _pad_rows(hx2[:, MY:]), zacc)
    out = _post_call(acc2[:, :N], REPD2, b2.reshape(1, -1))
    return out


# speedup vs baseline: 1.6499x; 1.2111x over previous
import functools

import jax
import jax.numpy as jnp
from jax import lax
from jax.experimental import pallas as pl
from jax.experimental.pallas import tpu as pltpu
from jax.experimental.pallas import tpu_sc as plsc

N = 10000
E = 320000
M = 128
HID = 16
HEADS = 8
MY = 64

ROW_BLOCK = 1000

NPAD = 10112
EBLK = 160
NWORK = 32
_NBLK0 = -(-E // (NWORK * EBLK))
NBLK = _NBLK0 + 1 - _NBLK0 % 2
EPAD = NWORK * EBLK * NBLK
NROW = NPAD // 16


def _pre_body(x_ref, w_ref, asdw_ref, hx_ref):
    h = x_ref[...] @ w_ref[...]
    hx_ref[...] = jnp.concatenate([h, h @ asdw_ref[...]], axis=1)


def _pre_call(x, W, ASDW):
    n, m = x.shape
    k = W.shape[1]
    cp = k + ASDW.shape[1]
    grid = n // ROW_BLOCK
    return pl.pallas_call(
        _pre_body,
        grid=(grid,),
        in_specs=[
            pl.BlockSpec((ROW_BLOCK, m), lambda i: (i, 0)),
            pl.BlockSpec((m, k), lambda i: (0, 0)),
            pl.BlockSpec((k, ASDW.shape[1]), lambda i: (0, 0)),
        ],
        out_specs=pl.BlockSpec((ROW_BLOCK, cp), lambda i: (i, 0)),
        out_shape=jax.ShapeDtypeStruct((n, cp), jnp.float32),
    )(x, W, ASDW)


def _mid_body(acca_ref, accb_ref, rda_ref, rdb_ref, b_ref, w_ref, asdw_ref,
              hx_ref):
    blka = acca_ref[0] + acca_ref[1]
    blkb = accb_ref[0] + accb_ref[1]
    c = rda_ref.shape[1]
    den = blka @ rda_ref[...] + blkb @ rdb_ref[...]
    num = jnp.concatenate([blka[:, : c // 2], blkb[:, : c // 2]], axis=1)
    act = num / (den + 1e-16) + b_ref[...]
    act = jnp.where(act > 0, act, jnp.exp(jnp.minimum(act, 0.0)) - 1.0)
    h = act @ w_ref[...]
    hx_ref[...] = jnp.concatenate([h, h @ asdw_ref[...]], axis=1)


def _mid_call(acca, accb, RDA, RDB, b, W, ASDW):
    _, n, cp = acca.shape
    k = W.shape[1]
    kp = k + ASDW.shape[1]
    grid = n // ROW_BLOCK
    return pl.pallas_call(
        _mid_body,
        grid=(grid,),
        in_specs=[
            pl.BlockSpec((2, ROW_BLOCK, cp), lambda i: (0, i, 0)),
            pl.BlockSpec((2, ROW_BLOCK, cp), lambda i: (0, i, 0)),
            pl.BlockSpec((cp, RDA.shape[1]), lambda i: (0, 0)),
            pl.BlockSpec((cp, RDA.shape[1]), lambda i: (0, 0)),
            pl.BlockSpec((1, RDA.shape[1]), lambda i: (0, 0)),
            pl.BlockSpec((RDA.shape[1], k), lambda i: (0, 0)),
            pl.BlockSpec((k, ASDW.shape[1]), lambda i: (0, 0)),
        ],
        out_specs=pl.BlockSpec((ROW_BLOCK, kp), lambda i: (i, 0)),
        out_shape=jax.ShapeDtypeStruct((n, kp), jnp.float32),
    )(acca, accb, RDA, RDB, b, W, ASDW)


def _post_body(acc_ref, repd_ref, b_ref, out_ref):
    blk = acc_ref[0] + acc_ref[1]
    c = repd_ref.shape[1]
    den = blk @ repd_ref[...]
    out_ref[...] = blk[:, :c] / (den + 1e-16) + b_ref[...]


def _post_call(acc, REPD, b):
    _, n, cp = acc.shape
    c = REPD.shape[1]
    grid = n // ROW_BLOCK
    return pl.pallas_call(
        _post_body,
        grid=(grid,),
        in_specs=[
            pl.BlockSpec((2, ROW_BLOCK, cp), lambda i: (0, i, 0)),
            pl.BlockSpec((cp, c), lambda i: (0, 0)),
            pl.BlockSpec((1, c), lambda i: (0, 0)),
        ],
        out_specs=pl.BlockSpec((ROW_BLOCK, c), lambda i: (i, 0)),
        out_shape=jax.ShapeDtypeStruct((n, c), jnp.float32),
    )(acc, REPD, b)


def _make_edge_kernel(H, C):
    CH = C // H
    CP = C + 8
    mesh = plsc.VectorSubcoreMesh(core_axis_name="c", subcore_axis_name="s")

    @functools.partial(
        pl.kernel,
        out_type=jax.ShapeDtypeStruct((2, NPAD, CP), jnp.float32),
        mesh=mesh,
        scratch_types=[
            pltpu.VMEM((NBLK, EBLK), jnp.int32),
            pltpu.VMEM((NBLK, EBLK), jnp.int32),
            pltpu.VMEM((2, EBLK, CP), jnp.float32),
            pltpu.VMEM((2, EBLK, 8), jnp.float32),
            pltpu.VMEM((2, EBLK, CP), jnp.float32),
            pltpu.VMEM_SHARED((NPAD, CP), jnp.float32),
            pltpu.SemaphoreType.DMA,
            pltpu.SemaphoreType.DMA,
            pltpu.SemaphoreType.DMA,
            pltpu.SemaphoreType.DMA,
            pltpu.SemaphoreType.DMA,
            pltpu.SemaphoreType.DMA,
        ],
        compiler_params=pltpu.CompilerParams(
            needs_layout_passes=False, use_tc_tiling_on_sc=False),
    )
    def ek(src_hbm, dst_hbm, hx_hbm, att_hbm, zacc_hbm, acc_out,
           idx_s, idx_d, hxb, attb, msg, acc,
           sem_h0, sem_h1, sem_a0, sem_a1, sem_w0, sem_w1):
        sem_h = (sem_h0, sem_h1)
        sem_a = (sem_a0, sem_a1)
        sem_w = (sem_w0, sem_w1)
        cid = lax.axis_index("c")
        sid = lax.axis_index("s")
        wid = cid * 16 + sid

        pltpu.sync_copy(zacc_hbm.at[pl.ds(sid * NROW, NROW)],
                        acc.at[pl.ds(sid * NROW, NROW)])
        plsc.subcore_barrier()

        pltpu.sync_copy(src_hbm.at[wid], idx_s)
        pltpu.sync_copy(dst_hbm.at[wid], idx_d)

        lanes = lax.iota(jnp.int32, 16)

        pltpu.sync_copy(zacc_hbm.at[pl.ds(0, EBLK)], msg.at[0])
        pltpu.sync_copy(zacc_hbm.at[pl.ds(0, EBLK)], msg.at[1])

        def gather(slot, b):
            pltpu.async_copy(hx_hbm.at[idx_s.at[b]], hxb.at[slot],
                             sem_h[slot])
            pltpu.async_copy(att_hbm.at[idx_d.at[b]], attb.at[slot],
                             sem_a[slot])

        def gather_wait(slot, b):
            pltpu.make_async_copy(hx_hbm.at[idx_s.at[b]], hxb.at[slot],
                                  sem_h[slot]).wait()
            pltpu.make_async_copy(att_hbm.at[idx_d.at[b]], attb.at[slot],
                                  sem_a[slot]).wait()

        def scatter(slot, b):
            pltpu.async_copy(msg.at[slot], acc.at[idx_d.at[b]],
                             sem_w[slot], add=True)

        def scatter_wait(slot, b):
            pltpu.make_async_copy(msg.at[slot], acc.at[idx_d.at[b]],
                                  sem_w[slot]).wait()

        def compute(slot):
            @plsc.parallel_loop(0, EBLK // 16, unroll=2)
            def _(g):
                row = lanes + g * 16
                for h in range(H):
                    sv = plsc.load_gather(
                        hxb.at[slot], [row, jnp.full((16,), C + h, jnp.int32)])
                    dv = plsc.load_gather(
                        attb.at[slot], [row, jnp.full((16,), 7 - h, jnp.int32)])
                    ev = sv + dv
                    w = jnp.exp(jnp.maximum(ev, 0.2 * ev))
                    plsc.store_scatter(
                        msg.at[slot], [row, jnp.full((16,), C + h, jnp.int32)], w)
                    for c in range(CH):
                        col = jnp.full((16,), h * CH + c, jnp.int32)
                        hv = plsc.load_gather(hxb.at[slot], [row, col])
                        plsc.store_scatter(msg.at[slot], [row, col], hv * w)

        gather(0, 0)

        def pair(i, _):
            b0 = 2 * i
            gather(1, b0 + 1)
            gather_wait(0, b0)

            @pl.when(i > 0)
            def _():
                scatter_wait(0, b0 - 2)

            compute(0)
            scatter(0, b0)
            gather(0, b0 + 2)
            gather_wait(1, b0 + 1)

            @pl.when(i > 0)
            def _():
                scatter_wait(1, b0 - 1)

            compute(1)
            scatter(1, b0 + 1)
            return 0

        lax.fori_loop(0, (NBLK - 1) // 2, pair, 0)
        gather_wait(0, NBLK - 1)
        scatter_wait(0, NBLK - 3)
        compute(0)
        scatter(0, NBLK - 1)
        scatter_wait(0, NBLK - 1)
        scatter_wait(1, NBLK - 2)

        plsc.subcore_barrier()
        pltpu.sync_copy(acc.at[pl.ds(sid * NROW, NROW)],
                        acc_out.at[cid, pl.ds(sid * NROW, NROW)])

    return ek


_edge_l1h = _make_edge_kernel(4, 64)
_edge_l2 = _make_edge_kernel(1, MY)


def _pad_rows(a):
    return jnp.pad(a, ((0, NPAD - N), (0, 0)))


def kernel(x, edge_index, W1, att_src1, att_dst1, b1, W2, att_src2, att_dst2, b2):
    src = edge_index[0].astype(jnp.int32)
    dst = edge_index[1].astype(jnp.int32)
    pad = jnp.full((EPAD - E,), N, jnp.int32)
    srcp = jnp.concatenate([src, pad]).reshape(NWORK, NBLK, EBLK)
    dstp = jnp.concatenate([dst, pad]).reshape(NWORK, NBLK, EBLK)

    eye8 = jnp.eye(HEADS, dtype=jnp.float32)
    AS1 = (att_src1[:, :, None] * eye8[:, None, :]).reshape(HEADS * HID, HEADS)
    AD1 = (att_dst1[:, :, None] * eye8[:, None, :]).reshape(HEADS * HID, HEADS)
    ATTA = jnp.concatenate([AS1[:, 0:4], AD1[:, 3::-1]], axis=1)
    ATTB = jnp.concatenate([AS1[:, 4:8], AD1[:, 7:3:-1]], axis=1)
    ASDW1 = jnp.concatenate([ATTA, ATTB], axis=1)
    ASDW2 = jnp.concatenate(
        [att_src2.T, jnp.zeros((MY, 6), jnp.float32), att_dst2.T], axis=1
    )
    REP1 = (eye8[:, :, None] * jnp.ones((1, 1, HID))).reshape(HEADS, HEADS * HID)
    z64 = jnp.zeros((MY, HEADS * HID), jnp.float32)
    z4 = jnp.zeros((4, HEADS * HID), jnp.float32)
    RDA = jnp.concatenate([z64, REP1[0:4], z4], axis=0)
    RDB = jnp.concatenate([z64, REP1[4:8], z4], axis=0)
    REPD2 = jnp.concatenate(
        [jnp.zeros((MY, MY), jnp.float32),
         jnp.zeros((8, MY), jnp.float32).at[0, :].set(1.0)], axis=0)

    zacc = jnp.zeros((NPAD, MY + 8), jnp.float32)

    hx1 = _pre_call(x, W1, ASDW1)
    hxa = _pad_rows(jnp.concatenate([hx1[:, 0:64], hx1[:, 128:136]], axis=1))
    hxb = _pad_rows(jnp.concatenate([hx1[:, 64:128], hx1[:, 136:144]], axis=1))
    acca = _edge_l1h(srcp, dstp, hxa, hxa[:, 64:], zacc)
    accb = _edge_l1h(srcp, dstp, hxb, hxb[:, 64:], zacc)
    hx2 = _mid_call(acca[:, :N], accb[:, :N], RDA, RDB, b1.reshape(1, -1),
                    W2, ASDW2)
    acc2 = _edge_l2(srcp, dstp, _pad_rows(hx2), _pad_rows(hx2[:, MY:]), zacc)
    out = _post_call(acc2[:, :N], REPD2, b2.reshape(1, -1))
    return out


# speedup vs baseline: 1.6587x; 1.0054x over previous
import functools

import jax
import jax.numpy as jnp
from jax import lax
from jax.experimental import pallas as pl
from jax.experimental.pallas import tpu as pltpu
from jax.experimental.pallas import tpu_sc as plsc

N = 10000
E = 320000
M = 128
HID = 16
HEADS = 8
MY = 64

ROW_BLOCK = 1000

NPAD = 10112
EBLK = 176
NWORK = 32
_NBLK0 = -(-E // (NWORK * EBLK))
NBLK = _NBLK0 + 1 - _NBLK0 % 2
EPAD = NWORK * EBLK * NBLK
NROW = NPAD // 16


def _pre_body(x_ref, w_ref, asdw_ref, hx_ref):
    h = x_ref[...] @ w_ref[...]
    hx_ref[...] = jnp.concatenate([h, h @ asdw_ref[...]], axis=1)


def _pre_call(x, W, ASDW):
    n, m = x.shape
    k = W.shape[1]
    cp = k + ASDW.shape[1]
    grid = n // ROW_BLOCK
    return pl.pallas_call(
        _pre_body,
        grid=(grid,),
        in_specs=[
            pl.BlockSpec((ROW_BLOCK, m), lambda i: (i, 0)),
            pl.BlockSpec((m, k), lambda i: (0, 0)),
            pl.BlockSpec((k, ASDW.shape[1]), lambda i: (0, 0)),
        ],
        out_specs=pl.BlockSpec((ROW_BLOCK, cp), lambda i: (i, 0)),
        out_shape=jax.ShapeDtypeStruct((n, cp), jnp.float32),
    )(x, W, ASDW)


def _mid_body(acca_ref, accb_ref, rda_ref, rdb_ref, b_ref, w_ref, asdw_ref,
              hx_ref):
    blka = acca_ref[0] + acca_ref[1]
    blkb = accb_ref[0] + accb_ref[1]
    c = rda_ref.shape[1]
    den = blka @ rda_ref[...] + blkb @ rdb_ref[...]
    num = jnp.concatenate([blka[:, : c // 2], blkb[:, : c // 2]], axis=1)
    act = num / (den + 1e-16) + b_ref[...]
    act = jnp.where(act > 0, act, jnp.exp(jnp.minimum(act, 0.0)) - 1.0)
    h = act @ w_ref[...]
    hx_ref[...] = jnp.concatenate([h, h @ asdw_ref[...]], axis=1)


def _mid_call(acca, accb, RDA, RDB, b, W, ASDW):
    _, n, cp = acca.shape
    k = W.shape[1]
    kp = k + ASDW.shape[1]
    grid = n // ROW_BLOCK
    return pl.pallas_call(
        _mid_body,
        grid=(grid,),
        in_specs=[
            pl.BlockSpec((2, ROW_BLOCK, cp), lambda i: (0, i, 0)),
            pl.BlockSpec((2, ROW_BLOCK, cp), lambda i: (0, i, 0)),
            pl.BlockSpec((cp, RDA.shape[1]), lambda i: (0, 0)),
            pl.BlockSpec((cp, RDA.shape[1]), lambda i: (0, 0)),
            pl.BlockSpec((1, RDA.shape[1]), lambda i: (0, 0)),
            pl.BlockSpec((RDA.shape[1], k), lambda i: (0, 0)),
            pl.BlockSpec((k, ASDW.shape[1]), lambda i: (0, 0)),
        ],
        out_specs=pl.BlockSpec((ROW_BLOCK, kp), lambda i: (i, 0)),
        out_shape=jax.ShapeDtypeStruct((n, kp), jnp.float32),
    )(acca, accb, RDA, RDB, b, W, ASDW)


def _post_body(acc_ref, repd_ref, b_ref, out_ref):
    blk = acc_ref[0] + acc_ref[1]
    c = repd_ref.shape[1]
    den = blk @ repd_ref[...]
    out_ref[...] = blk[:, :c] / (den + 1e-16) + b_ref[...]


def _post_call(acc, REPD, b):
    _, n, cp = acc.shape
    c = REPD.shape[1]
    grid = n // ROW_BLOCK
    return pl.pallas_call(
        _post_body,
        grid=(grid,),
        in_specs=[
            pl.BlockSpec((2, ROW_BLOCK, cp), lambda i: (0, i, 0)),
            pl.BlockSpec((cp, c), lambda i: (0, 0)),
            pl.BlockSpec((1, c), lambda i: (0, 0)),
        ],
        out_specs=pl.BlockSpec((ROW_BLOCK, c), lambda i: (i, 0)),
        out_shape=jax.ShapeDtypeStruct((n, c), jnp.float32),
    )(acc, REPD, b)


def _make_edge_kernel(H, C):
    CH = C // H
    CP = C + 8
    mesh = plsc.VectorSubcoreMesh(core_axis_name="c", subcore_axis_name="s")

    @functools.partial(
        pl.kernel,
        out_type=jax.ShapeDtypeStruct((2, NPAD, CP), jnp.float32),
        mesh=mesh,
        scratch_types=[
            pltpu.VMEM((NBLK, EBLK), jnp.int32),
            pltpu.VMEM((NBLK, EBLK), jnp.int32),
            pltpu.VMEM((2, EBLK, CP), jnp.float32),
            pltpu.VMEM((2, EBLK, 8), jnp.float32),
            pltpu.VMEM((2, EBLK, CP), jnp.float32),
            pltpu.VMEM_SHARED((NPAD, CP), jnp.float32),
            pltpu.SemaphoreType.DMA,
            pltpu.SemaphoreType.DMA,
            pltpu.SemaphoreType.DMA,
            pltpu.SemaphoreType.DMA,
            pltpu.SemaphoreType.DMA,
            pltpu.SemaphoreType.DMA,
        ],
        compiler_params=pltpu.CompilerParams(
            needs_layout_passes=False, use_tc_tiling_on_sc=False),
    )
    def ek(src_hbm, dst_hbm, hx_hbm, att_hbm, zacc_hbm, acc_out,
           idx_s, idx_d, hxb, attb, msg, acc,
           sem_h0, sem_h1, sem_a0, sem_a1, sem_w0, sem_w1):
        sem_h = (sem_h0, sem_h1)
        sem_a = (sem_a0, sem_a1)
        sem_w = (sem_w0, sem_w1)
        cid = lax.axis_index("c")
        sid = lax.axis_index("s")
        wid = cid * 16 + sid

        pltpu.sync_copy(zacc_hbm.at[pl.ds(sid * NROW, NROW)],
                        acc.at[pl.ds(sid * NROW, NROW)])
        plsc.subcore_barrier()

        pltpu.sync_copy(src_hbm.at[wid], idx_s)
        pltpu.sync_copy(dst_hbm.at[wid], idx_d)

        lanes = lax.iota(jnp.int32, 16)

        pltpu.sync_copy(zacc_hbm.at[pl.ds(0, EBLK)], msg.at[0])
        pltpu.sync_copy(zacc_hbm.at[pl.ds(0, EBLK)], msg.at[1])

        def gather(slot, b):
            pltpu.async_copy(hx_hbm.at[idx_s.at[b]], hxb.at[slot],
                             sem_h[slot])
            pltpu.async_copy(att_hbm.at[idx_d.at[b]], attb.at[slot],
                             sem_a[slot])

        def gather_wait(slot, b):
            pltpu.make_async_copy(hx_hbm.at[idx_s.at[b]], hxb.at[slot],
                                  sem_h[slot]).wait()
            pltpu.make_async_copy(att_hbm.at[idx_d.at[b]], attb.at[slot],
                                  sem_a[slot]).wait()

        def scatter(slot, b):
            pltpu.async_copy(msg.at[slot], acc.at[idx_d.at[b]],
                             sem_w[slot], add=True)

        def scatter_wait(slot, b):
            pltpu.make_async_copy(msg.at[slot], acc.at[idx_d.at[b]],
                                  sem_w[slot]).wait()

        def compute(slot):
            @plsc.parallel_loop(0, EBLK // 16, unroll=2)
            def _(g):
                row = lanes + g * 16
                for h in range(H):
                    sv = plsc.load_gather(
                        hxb.at[slot], [row, jnp.full((16,), C + h, jnp.int32)])
                    dv = plsc.load_gather(
                        attb.at[slot], [row, jnp.full((16,), 7 - h, jnp.int32)])
                    ev = sv + dv
                    w = jnp.exp(jnp.maximum(ev, 0.2 * ev))
                    plsc.store_scatter(
                        msg.at[slot], [row, jnp.full((16,), C + h, jnp.int32)], w)
                    for c in range(CH):
                        col = jnp.full((16,), h * CH + c, jnp.int32)
                        hv = plsc.load_gather(hxb.at[slot], [row, col])
                        plsc.store_scatter(msg.at[slot], [row, col], hv * w)

        gather(0, 0)

        def pair(i, _):
            b0 = 2 * i
            gather(1, b0 + 1)
            gather_wait(0, b0)

            @pl.when(i > 0)
            def _():
                scatter_wait(0, b0 - 2)

            compute(0)
            scatter(0, b0)
            gather(0, b0 + 2)
            gather_wait(1, b0 + 1)

            @pl.when(i > 0)
            def _():
                scatter_wait(1, b0 - 1)

            compute(1)
            scatter(1, b0 + 1)
            return 0

        lax.fori_loop(0, (NBLK - 1) // 2, pair, 0)
        gather_wait(0, NBLK - 1)
        scatter_wait(0, NBLK - 3)
        compute(0)
        scatter(0, NBLK - 1)
        scatter_wait(0, NBLK - 1)
        scatter_wait(1, NBLK - 2)

        plsc.subcore_barrier()
        pltpu.sync_copy(acc.at[pl.ds(sid * NROW, NROW)],
                        acc_out.at[cid, pl.ds(sid * NROW, NROW)])

    return ek


_edge_l1h = _make_edge_kernel(4, 64)
_edge_l2 = _make_edge_kernel(1, MY)


def _pad_rows(a):
    return jnp.pad(a, ((0, NPAD - N), (0, 0)))


def kernel(x, edge_index, W1, att_src1, att_dst1, b1, W2, att_src2, att_dst2, b2):
    src = edge_index[0].astype(jnp.int32)
    dst = edge_index[1].astype(jnp.int32)
    pad = jnp.full((EPAD - E,), N, jnp.int32)
    srcp = jnp.concatenate([src, pad]).reshape(NWORK, NBLK, EBLK)
    dstp = jnp.concatenate([dst, pad]).reshape(NWORK, NBLK, EBLK)

    eye8 = jnp.eye(HEADS, dtype=jnp.float32)
    AS1 = (att_src1[:, :, None] * eye8[:, None, :]).reshape(HEADS * HID, HEADS)
    AD1 = (att_dst1[:, :, None] * eye8[:, None, :]).reshape(HEADS * HID, HEADS)
    ATTA = jnp.concatenate([AS1[:, 0:4], AD1[:, 3::-1]], axis=1)
    ATTB = jnp.concatenate([AS1[:, 4:8], AD1[:, 7:3:-1]], axis=1)
    ASDW1 = jnp.concatenate([ATTA, ATTB], axis=1)
    ASDW2 = jnp.concatenate(
        [att_src2.T, jnp.zeros((MY, 6), jnp.float32), att_dst2.T], axis=1
    )
    REP1 = (eye8[:, :, None] * jnp.ones((1, 1, HID))).reshape(HEADS, HEADS * HID)
    z64 = jnp.zeros((MY, HEADS * HID), jnp.float32)
    z4 = jnp.zeros((4, HEADS * HID), jnp.float32)
    RDA = jnp.concatenate([z64, REP1[0:4], z4], axis=0)
    RDB = jnp.concatenate([z64, REP1[4:8], z4], axis=0)
    REPD2 = jnp.concatenate(
        [jnp.zeros((MY, MY), jnp.float32),
         jnp.zeros((8, MY), jnp.float32).at[0, :].set(1.0)], axis=0)

    zacc = jnp.zeros((NPAD, MY + 8), jnp.float32)

    hx1 = _pre_call(x, W1, ASDW1)
    hxa = _pad_rows(jnp.concatenate([hx1[:, 0:64], hx1[:, 128:136]], axis=1))
    hxb = _pad_rows(jnp.concatenate([hx1[:, 64:128], hx1[:, 136:144]], axis=1))
    acca = _edge_l1h(srcp, dstp, hxa, hxa[:, 64:], zacc)
    accb = _edge_l1h(srcp, dstp, hxb, hxb[:, 64:], zacc)
    hx2 = _mid_call(acca[:, :N], accb[:, :N], RDA, RDB, b1.reshape(1, -1),
                    W2, ASDW2)
    acc2 = _edge_l2(srcp, dstp, _pad_rows(hx2), _pad_rows(hx2[:, MY:]), zacc)
    out = _post_call(acc2[:, :N], REPD2, b2.reshape(1, -1))
    return out
